# Initial kernel scaffold; baseline (speedup 1.0000x reference)
import jax
import jax.numpy as jnp
from jax.experimental import pallas as pl


def kernel(W0, W1, W2, inputs):
    raise NotImplementedError("write your pallas kernel here")


# speedup vs baseline: 1.5174x; 1.5174x over previous
import functools

import jax
import jax.numpy as jnp
from jax import lax
from jax.experimental import pallas as pl
from jax.experimental.pallas import tpu as pltpu
from jax.experimental.pallas import tpu_sc as plsc

_VOCAB = 100000
_DIM = 768
_B = 4 * 2048
_NC = 2
_NS = 16
_NW = _NC * _NS
_BPW = _B // _NW
_CHUNK = 64
_NCHUNK = _BPW // _CHUNK


@jax.jit
def _sc_gather3(W0, W1, W2, idx_flat):
    mesh = plsc.VectorSubcoreMesh(
        core_axis_name="c", subcore_axis_name="s", num_cores=_NC,
        num_subcores=_NS)
    out_type = [jax.ShapeDtypeStruct((_B, _DIM), jnp.float32)] * 3

    @functools.partial(
        pl.kernel,
        mesh=mesh,
        out_type=out_type,
        scratch_types=[
            pltpu.VMEM((_BPW,), jnp.int32),
            pltpu.VMEM((_CHUNK, _DIM), jnp.float32),
            pltpu.VMEM((_CHUNK, _DIM), jnp.float32),
            pltpu.SemaphoreType.DMA,
            pltpu.SemaphoreType.DMA,
        ],
    )
    def body(w0, w1, w2, idx_hbm, o0, o1, o2, idx_v, buf0, buf1, sem0, sem1):
        wid = lax.axis_index("s") * _NC + lax.axis_index("c")
        base = wid * _BPW
        pltpu.sync_copy(idx_hbm.at[pl.ds(base, _BPW)], idx_v)

        tables = (w0, w1, w2)
        outs = (o0, o1, o2)
        bufs = (buf0, buf1)
        sems = (sem0, sem1)
        tasks = [(t, c) for t in range(3) for c in range(_NCHUNK)]

        def start(i):
            t, c = tasks[i]
            return pltpu.async_copy(
                tables[t].at[idx_v.at[pl.ds(c * _CHUNK, _CHUNK)]],
                bufs[i % 2], sems[i % 2])

        cp = start(0)
        for i, (t, c) in enumerate(tasks):
            nxt = start(i + 1) if i + 1 < len(tasks) else None
            cp.wait()
            pltpu.sync_copy(bufs[i % 2],
                            outs[t].at[pl.ds(base + c * _CHUNK, _CHUNK)])
            cp = nxt

    return body(W0, W1, W2, idx_flat)


def kernel(W0, W1, W2, inputs):
    idx_flat = inputs.reshape(-1).astype(jnp.int32)
    e0, e1, e2 = _sc_gather3(W0, W1, W2, idx_flat)
    shape = inputs.shape + (_DIM,)
    e0 = e0.reshape(shape)
    e1 = e1.reshape(shape)
    e2 = e2.reshape(shape)
    return (e0, e1, e2, e2, e1, e0)
